# baseline (speedup 1.0000x reference)
import functools

import jax
import jax.numpy as jnp
from jax import lax
from jax.experimental import pallas as pl
from jax.experimental.pallas import tpu as pltpu
from jax.experimental.pallas import tpu_sc as plsc

B, C, H, W = 128, 768, 14, 14
E = 64
TOP_K = 8
HW = H * W
TAIL = 56
HEAD = HW - TAIL
HW_BLK = 14
NEG = -3.0e38

N_CORES = 2
N_SUBCORES = 16
N_WORKERS = N_CORES * N_SUBCORES
RPW = B // N_WORKERS
CH = 8
N_CHUNKS = TAIL // CH
LANES_PER_ROW = C // 16


def _head_pool_body(x_ref, out_ref, acc_ref):
    i = pl.program_id(0)

    @pl.when(i == 0)
    def _init():
        acc_ref[...] = jnp.zeros_like(acc_ref)

    acc_ref[...] += jnp.sum(x_ref[...], axis=0)

    @pl.when(i == pl.num_programs(0) - 1)
    def _out():
        out_ref[...] = acc_ref[...]


def _sc_tail_pool_body(xs_hbm, out_hbm, buf, acc):
    wid = lax.axis_index("s") * N_CORES + lax.axis_index("c")
    base = wid * RPW

    def _zero(i, _):
        acc[0, pl.ds(i * 16, 16)] = jnp.zeros((16,), jnp.float32)
        acc[1, pl.ds(i * 16, 16)] = jnp.zeros((16,), jnp.float32)
        acc[2, pl.ds(i * 16, 16)] = jnp.zeros((16,), jnp.float32)
        acc[3, pl.ds(i * 16, 16)] = jnp.zeros((16,), jnp.float32)
        return 0
    lax.fori_loop(0, LANES_PER_ROW, _zero, 0)

    def _chunk(c, _):
        pltpu.sync_copy(
            xs_hbm.at[pl.ds(HEAD + c * CH, CH), pl.ds(base, RPW)], buf)

        def _lane(i, _):
            off = i * 16
            for r in range(RPW):
                s = acc[r, pl.ds(off, 16)]
                for t in range(CH):
                    s = s + buf[t, r, pl.ds(off, 16)]
                acc[r, pl.ds(off, 16)] = s
            return 0
        lax.fori_loop(0, LANES_PER_ROW, _lane, 0)
        return 0
    lax.fori_loop(0, N_CHUNKS, _chunk, 0)

    pltpu.sync_copy(acc, out_hbm.at[pl.ds(base, RPW)])


_sc_tail_pool = functools.partial(
    pl.kernel,
    out_type=jax.ShapeDtypeStruct((B, C), jnp.float32),
    mesh=plsc.VectorSubcoreMesh(core_axis_name="c", subcore_axis_name="s"),
    compiler_params=pltpu.CompilerParams(needs_layout_passes=False),
    scratch_types=[
        pltpu.VMEM((CH, RPW, C), jnp.float32),
        pltpu.VMEM((RPW, C), jnp.float32),
    ],
)(_sc_tail_pool_body)


def _combine_gate_body(a_ref, b_ref, w_ref, out_ref):
    pooled = (a_ref[...] + b_ref[...]) * jnp.float32(1.0 / HW)
    logits = jnp.dot(pooled, w_ref[...],
                     preferred_element_type=jnp.float32)
    cols = jax.lax.broadcasted_iota(jnp.int32, (B, E), 1)
    selected = jnp.zeros((B, E), dtype=jnp.bool_)
    avail = logits
    for _ in range(TOP_K):
        m = jnp.max(avail, axis=1, keepdims=True)
        cand = avail == m
        idx = jnp.min(jnp.where(cand, cols, E), axis=1, keepdims=True)
        first = cand & (cols == idx)
        selected = selected | first
        avail = jnp.where(first, NEG, avail)
    mx = jnp.max(jnp.where(selected, logits, NEG), axis=1, keepdims=True)
    ex = jnp.where(selected, jnp.exp(logits - mx), jnp.float32(0.0))
    out_ref[...] = ex / jnp.sum(ex, axis=1, keepdims=True)


@jax.jit
def kernel(x, W_gate):
    xs = jnp.transpose(x, (2, 3, 0, 1)).reshape(HW, B, C)
    part_sc = _sc_tail_pool(xs)
    part_tc = pl.pallas_call(
        _head_pool_body,
        grid=(HEAD // HW_BLK,),
        in_specs=[pl.BlockSpec((HW_BLK, B, C), lambda i: (i, 0, 0))],
        out_specs=pl.BlockSpec((B, C), lambda i: (0, 0)),
        out_shape=jax.ShapeDtypeStruct((B, C), jnp.float32),
        scratch_shapes=[pltpu.VMEM((B, C), jnp.float32)],
    )(xs)
    return pl.pallas_call(
        _combine_gate_body,
        in_specs=[
            pl.BlockSpec((B, C), lambda: (0, 0)),
            pl.BlockSpec((B, C), lambda: (0, 0)),
            pl.BlockSpec((C, E), lambda: (0, 0)),
        ],
        out_specs=pl.BlockSpec((B, E), lambda: (0, 0)),
        out_shape=jax.ShapeDtypeStruct((B, E), jnp.float32),
    )(part_tc, part_sc, W_gate)
